# baseline (speedup 1.0000x reference)
import functools

import jax
import jax.numpy as jnp
from jax import lax
from jax.experimental import pallas as pl
from jax.experimental.pallas import tpu as pltpu
from jax.experimental.pallas import tpu_sc as plsc

N = 10000
E = 320000
D1 = 128
H1 = 8
C1 = 16
D2 = 16
R1 = D1 + 16
R2 = D2 + 16

NCORES = 2
NSUB = 16
NTILE = NCORES * NSUB
EPT = E // NTILE
K = 80
NCH = EPT // K
NPAD = 10240
RPT = NPAD // NSUB

BN = 2000
NB = N // BN
BIG = 1e30


def _tc1_body(x_ref, w1_ref, a1s_ref, a1d_ref,
              hs_ref, abd_ref, cvec_ref):
    i = pl.program_id(0)
    h = jnp.dot(x_ref[...], w1_ref[...], preferred_element_type=jnp.float32)
    s = jnp.dot(h, a1s_ref[...], preferred_element_type=jnp.float32)
    d = jnp.dot(h, a1d_ref[...], preferred_element_type=jnp.float32)
    hs_ref[:, 0:D1] = h
    hs_ref[:, D1:R1] = s
    abd_ref[...] = d
    bm = jnp.max(s, axis=0, keepdims=True)

    @pl.when(i == 0)
    def _():
        cvec_ref[...] = bm

    @pl.when(i > 0)
    def _():
        cvec_ref[...] = jnp.maximum(cvec_ref[...], bm)

    @pl.when(i == NB - 1)
    def _():
        m = cvec_ref[...]
        c = m[:, 0:8] + m[:, 8:16]
        c = jnp.where(c >= 0.0, c, 0.2 * c)
        cvec_ref[...] = jnp.concatenate(
            [c, jnp.full((1, 8), BIG, jnp.float32)], axis=1)


def _tc1(x, W1, A1s, A1d):
    return pl.pallas_call(
        _tc1_body,
        grid=(NB,),
        in_specs=[
            pl.BlockSpec((BN, D1), lambda i: (i, 0)),
            pl.BlockSpec((D1, D1), lambda i: (0, 0)),
            pl.BlockSpec((D1, 16), lambda i: (0, 0)),
            pl.BlockSpec((D1, 16), lambda i: (0, 0)),
        ],
        out_specs=[
            pl.BlockSpec((BN, R1), lambda i: (i, 0)),
            pl.BlockSpec((BN, 16), lambda i: (i, 0)),
            pl.BlockSpec((1, 16), lambda i: (0, 0)),
        ],
        out_shape=[
            jax.ShapeDtypeStruct((N, R1), jnp.float32),
            jax.ShapeDtypeStruct((N, 16), jnp.float32),
            jax.ShapeDtypeStruct((1, 16), jnp.float32),
        ],
    )(x, W1, A1s, A1d)


def _tc2_body(acc_ref, b1_ref, w2_ref, p2s_ref, p2d_ref, q_ref,
              hs2_ref, abd_ref, cvec_ref, cm_ref):
    i = pl.program_id(0)
    num = acc_ref[0, :, 0:D1] + acc_ref[1, :, 0:D1]
    den = acc_ref[0, :, D1:R1] + acc_ref[1, :, D1:R1]
    dd = den + 1e-16
    r = 1.0 / dd
    r = r * (2.0 - dd * r)
    r = r * (2.0 - dd * r)
    rb = jnp.dot(r, q_ref[...], preferred_element_type=jnp.float32)
    feat = num * rb + b1_ref[...]
    feat = jnp.where(feat > 0.0, feat, jnp.exp(feat) - 1.0)
    h2 = jnp.dot(feat, w2_ref[...], preferred_element_type=jnp.float32)
    s = jnp.dot(h2, p2s_ref[...], preferred_element_type=jnp.float32)
    d = jnp.dot(h2, p2d_ref[...], preferred_element_type=jnp.float32)
    hs2_ref[:, 0:D2] = h2
    hs2_ref[:, D2:R2] = s
    abd_ref[...] = d
    bs = jnp.max(s, axis=0, keepdims=True)
    bd = jnp.max(d, axis=0, keepdims=True)

    @pl.when(i == 0)
    def _():
        cm_ref[0:1] = bs
        cm_ref[1:2] = bd

    @pl.when(i > 0)
    def _():
        cm_ref[0:1] = jnp.maximum(cm_ref[0:1], bs)
        cm_ref[1:2] = jnp.maximum(cm_ref[1:2], bd)

    @pl.when(i == NB - 1)
    def _():
        c = cm_ref[0:1] + cm_ref[1:2]
        c = jnp.where(c >= 0.0, c, 0.2 * c)
        lane = lax.broadcasted_iota(jnp.int32, (1, 16), 1)
        cvec_ref[...] = jnp.where(lane == 0, c, BIG)


def _tc2(acc, b1, W2, P2s, P2d, Q16):
    return pl.pallas_call(
        _tc2_body,
        grid=(NB,),
        in_specs=[
            pl.BlockSpec((2, BN, R1), lambda i: (0, i, 0)),
            pl.BlockSpec((1, D1), lambda i: (0, 0)),
            pl.BlockSpec((D1, D2), lambda i: (0, 0)),
            pl.BlockSpec((D2, 16), lambda i: (0, 0)),
            pl.BlockSpec((D2, 16), lambda i: (0, 0)),
            pl.BlockSpec((16, D1), lambda i: (0, 0)),
        ],
        out_specs=[
            pl.BlockSpec((BN, R2), lambda i: (i, 0)),
            pl.BlockSpec((BN, 16), lambda i: (i, 0)),
            pl.BlockSpec((1, 16), lambda i: (0, 0)),
        ],
        out_shape=[
            jax.ShapeDtypeStruct((N, R2), jnp.float32),
            jax.ShapeDtypeStruct((N, 16), jnp.float32),
            jax.ShapeDtypeStruct((1, 16), jnp.float32),
        ],
        scratch_shapes=[pltpu.VMEM((2, 16), jnp.float32)],
    )(acc, b1, W2, P2s, P2d, Q16)


def _tc3_body(acc_ref, b2_ref, out_ref):
    num = acc_ref[0, :, 0:D2] + acc_ref[1, :, 0:D2]
    den = acc_ref[0, :, D2:D2 + 1] + acc_ref[1, :, D2:D2 + 1]
    dd = den + 1e-16
    r = 1.0 / dd
    r = r * (2.0 - dd * r)
    r = r * (2.0 - dd * r)
    o = num * r + b2_ref[...]
    m = jnp.max(o, axis=1, keepdims=True)
    z = o - m
    lse = jnp.log(jnp.sum(jnp.exp(z), axis=1, keepdims=True))
    out_ref[...] = z - lse


def _tc3(acc, b2):
    return pl.pallas_call(
        _tc3_body,
        grid=(NB,),
        in_specs=[
            pl.BlockSpec((2, BN, R2), lambda i: (0, i, 0)),
            pl.BlockSpec((1, 16), lambda i: (0, 0)),
        ],
        out_specs=pl.BlockSpec((BN, 16), lambda i: (i, 0)),
        out_shape=jax.ShapeDtypeStruct((N, 16), jnp.float32),
    )(acc, b2)


_MESH = plsc.VectorSubcoreMesh(core_axis_name="c", subcore_axis_name="s")


def _make_edge_body(D, nheads):
    F = D - 16

    def body(hs_hbm, abd_hbm, esrc_hbm, dst2d_hbm, cvec_hbm, zD_hbm,
             acc_out,
             acc, dst_all, si0, si1, hs0, hs1, ad0, ad1, cvec_v,
             gh0, gd0, sn0, ix0, gh1, gd1, sn1, ix1):
        cid = lax.axis_index("c")
        sid = lax.axis_index("s")
        gid = cid * NSUB + sid
        r0 = sid * RPT
        pltpu.sync_copy(zD_hbm.at[pl.ds(r0, RPT)], acc.at[pl.ds(r0, RPT)])
        pltpu.sync_copy(cvec_hbm.at[0], cvec_v)
        pltpu.sync_copy(dst2d_hbm.at[pl.ds(gid * NCH, NCH)], dst_all)
        plsc.subcore_barrier()
        cv = cvec_v[...]

        bufs = ((hs0, ad0, si0, gh0, gd0, sn0, ix0),
                (hs1, ad1, si1, gh1, gd1, sn1, ix1))

        def issue_idx(b, j):
            _hs, _ad, si, _gh, _gd, _sn, ix = bufs[b]
            pltpu.async_copy(esrc_hbm.at[pl.ds(gid * EPT + j * K, K)], si, ix)

        def wait_idx(b, j):
            _hs, _ad, si, _gh, _gd, _sn, ix = bufs[b]
            pltpu.make_async_copy(
                esrc_hbm.at[pl.ds(gid * EPT + j * K, K)], si, ix).wait()

        def issue_g(b, j):
            hs_r, ad_r, si, gh, gd, _sn, _ix = bufs[b]
            pltpu.async_copy(hs_hbm.at[si], hs_r, gh)
            pltpu.async_copy(abd_hbm.at[dst_all.at[j]], ad_r, gd)

        def wait_g(b, j):
            hs_r, ad_r, si, gh, gd, _sn, _ix = bufs[b]
            pltpu.make_async_copy(hs_hbm.at[si], hs_r, gh).wait()
            pltpu.make_async_copy(abd_hbm.at[dst_all.at[j]], ad_r, gd).wait()

        def issue_s(b, j):
            hs_r, _ad, _si, _gh, _gd, sn, _ix = bufs[b]
            pltpu.async_copy(hs_r, acc.at[dst_all.at[j]], sn, add=True)

        def wait_s(b, j):
            hs_r, _ad, _si, _gh, _gd, sn, _ix = bufs[b]
            pltpu.make_async_copy(hs_r, acc.at[dst_all.at[j]], sn).wait()

        def compute(b):
            hs_r, ad_r = bufs[b][0], bufs[b][1]

            @plsc.parallel_loop(0, K, 1, unroll=2)
            def _(e):
                t = hs_r[e, pl.ds(F, 16)] + ad_r[e]
                t = jnp.where(t >= 0.0, t, t * 0.2)
                w = jnp.exp(t - cv)
                if nheads == 1:
                    sl = pl.ds(0, 16)
                    hs_r[e, sl] = hs_r[e, sl] * w[0]
                else:
                    for hh in range(nheads):
                        sl = pl.ds(hh * 16, 16)
                        hs_r[e, sl] = hs_r[e, sl] * w[hh]
                hs_r[e, pl.ds(F, 16)] = w

        issue_idx(0, 0)
        issue_idx(1, 1)
        wait_idx(0, 0)
        issue_g(0, 0)
        wait_idx(1, 1)
        issue_g(1, 1)

        def pair(jj, carry):
            wait_g(0, jj)

            @pl.when(jj + 2 < NCH)
            def _():
                issue_idx(0, jj + 2)

            compute(0)
            issue_s(0, jj)
            wait_g(1, jj + 1)

            @pl.when(jj + 3 < NCH)
            def _():
                issue_idx(1, jj + 3)

            compute(1)
            issue_s(1, jj + 1)

            @pl.when(jj + 2 < NCH)
            def _():
                wait_s(0, jj)
                wait_idx(0, jj + 2)
                issue_g(0, jj + 2)

            @pl.when(jj + 3 < NCH)
            def _():
                wait_s(1, jj + 1)
                wait_idx(1, jj + 3)
                issue_g(1, jj + 3)

            return carry

        lax.fori_loop(0, (NCH - 1) // 2, lambda i, c: pair(2 * i, c), 0)
        wait_g(0, NCH - 1)
        compute(0)
        issue_s(0, NCH - 1)
        wait_s(0, NCH - 1)
        wait_s(1, NCH - 2)
        plsc.subcore_barrier()
        pltpu.sync_copy(acc.at[pl.ds(r0, RPT)],
                        acc_out.at[cid, pl.ds(r0, RPT)])

    return body


def _make_edge(D, nheads):
    return functools.partial(
        pl.kernel,
        out_type=jax.ShapeDtypeStruct((NCORES, NPAD, D), jnp.float32),
        mesh=_MESH,
        compiler_params=pltpu.CompilerParams(use_tc_tiling_on_sc=False),
        scratch_types=[
            pltpu.VMEM_SHARED((NPAD, D), jnp.float32),
            pltpu.VMEM((NCH, K), jnp.int32),
            pltpu.VMEM((K,), jnp.int32),
            pltpu.VMEM((K,), jnp.int32),
            pltpu.VMEM((K, D), jnp.float32),
            pltpu.VMEM((K, D), jnp.float32),
            pltpu.VMEM((K, 16), jnp.float32),
            pltpu.VMEM((K, 16), jnp.float32),
            pltpu.VMEM((16,), jnp.float32),
        ] + [pltpu.SemaphoreType.DMA] * 8,
    )(_make_edge_body(D, nheads))


_edge1 = _make_edge(R1, H1)
_edge2 = _make_edge(R2, 1)


def kernel(x, edge_index, W1, att_src1, att_dst1, b1, W2, att_src2, att_dst2, b2):
    f32 = jnp.float32
    eye8 = jnp.eye(H1, dtype=f32)
    As = (eye8[:, None, :] * att_src1[:, :, None]).reshape(D1, H1)
    Ad = (eye8[:, None, :] * att_dst1[:, :, None]).reshape(D1, H1)
    A1s = jnp.concatenate([As, Ad], axis=1)
    A1d = jnp.concatenate([Ad, As], axis=1)
    P2s = jnp.concatenate(
        [att_src2.reshape(D2, 1), jnp.zeros((D2, 15), f32)], axis=1)
    P2d = jnp.concatenate(
        [att_dst2.reshape(D2, 1), jnp.zeros((D2, 15), f32)], axis=1)
    Q16 = jnp.concatenate(
        [jnp.kron(jnp.eye(H1, dtype=f32), jnp.ones((1, C1), f32)),
         jnp.zeros((8, D1), f32)], axis=0)
    zr1 = jnp.zeros((NPAD, R1), f32)
    zr2 = jnp.zeros((NPAD, R2), f32)

    hs1, ab1d, cvec1 = _tc1(x, W1, A1s, A1d)
    esrc = edge_index[0]
    dst2d = edge_index[1].reshape(E // K, K)
    acc1 = _edge1(hs1, ab1d, esrc, dst2d, cvec1, zr1)
    hs2, ab2d, cvec2 = _tc2(acc1, b1.reshape(1, D1), W2, P2s, P2d, Q16)
    acc2 = _edge2(hs2, ab2d, esrc, dst2d, cvec2, zr2)
    return _tc3(acc2, b2.reshape(1, 16))
